# Initial kernel scaffold; baseline (speedup 1.0000x reference)
import jax
import jax.numpy as jnp
from jax.experimental import pallas as pl


def kernel(guess_indices, constraint_indices, presence_list, absent_list, guess_table, constraint_table, W1, b1, W2, b2):
    raise NotImplementedError("write your pallas kernel here")


# speedup vs baseline: 24.1848x; 24.1848x over previous
import functools

import jax
import jax.numpy as jnp
from jax.experimental import pallas as pl

B = 16384
D = 8
H = 256
K = 384
BB = 2048


def _body(idx_ref, gt_ref, ct_ref, w1_ref, b1_ref, w2_ref, b2_ref, out_ref):
    gpad = jnp.concatenate([gt_ref[:], jnp.zeros((6, D), jnp.float32)], axis=0)
    cpad = jnp.concatenate([ct_ref[:], jnp.zeros((5, D), jnp.float32)], axis=0)
    w1 = w1_ref[:]
    parts = []
    for p in range(5):
        parts.append(jax.lax.dot(gpad, w1[8 * p:8 * p + 8, :],
                                 preferred_element_type=jnp.float32))
    for p in range(5):
        parts.append(jax.lax.dot(cpad, w1[40 + 8 * p:48 + 8 * p, :],
                                 preferred_element_type=jnp.float32))
    parts.append(jax.lax.dot(gpad, w1[80:88, :], preferred_element_type=jnp.float32))
    parts.append(jax.lax.dot(gpad, w1[88:96, :], preferred_element_type=jnp.float32))
    a = jnp.concatenate(parts, axis=0)

    idx = idx_ref[:]
    lane = jax.lax.broadcasted_iota(jnp.int32, (BB, 128), 1)

    c0 = jnp.zeros((BB, 128), jnp.float32)
    for p in range(4):
        c0 = c0 + jnp.where(lane == idx[:, p:p + 1] + 32 * p, 1.0, 0.0)
    c1 = jnp.where(lane == idx[:, 4:5], 1.0, 0.0)
    for p in range(3):
        c1 = c1 + jnp.where(lane == idx[:, 5 + p:6 + p] + 32 * (p + 1), 1.0, 0.0)
    c2 = jnp.where(lane == idx[:, 8:9], 1.0, 0.0)
    c2 = c2 + jnp.where(lane == idx[:, 9:10] + 32, 1.0, 0.0)
    for j in range(10):
        c2 = c2 + jnp.where(lane == idx[:, 10 + j:11 + j] + 64, 0.1, 0.0)
    for j in range(10):
        c2 = c2 + jnp.where(lane == idx[:, 20 + j:21 + j] + 96, 0.1, 0.0)
    o = jnp.concatenate([c0, c1, c2], axis=1)

    h = jax.lax.dot(o, a, preferred_element_type=jnp.float32) + b1_ref[:]
    h = jnp.maximum(h, 0.0)
    out = jax.lax.dot(h, w2_ref[:], preferred_element_type=jnp.float32)
    out_ref[:] = out + b2_ref[:]


@jax.jit
def kernel(guess_indices, constraint_indices, presence_list, absent_list,
           guess_table, constraint_table, W1, b1, W2, b2):
    idx = jnp.concatenate([guess_indices, constraint_indices,
                           presence_list, absent_list], axis=1).astype(jnp.int32)
    b1r = b1.reshape(1, H)
    b2r = b2.reshape(1, 1)
    grid = (B // BB,)
    return pl.pallas_call(
        _body,
        grid=grid,
        in_specs=[
            pl.BlockSpec((BB, 30), lambda i: (i, 0)),
            pl.BlockSpec((26, D), lambda i: (0, 0)),
            pl.BlockSpec((27, D), lambda i: (0, 0)),
            pl.BlockSpec((96, H), lambda i: (0, 0)),
            pl.BlockSpec((1, H), lambda i: (0, 0)),
            pl.BlockSpec((H, 1), lambda i: (0, 0)),
            pl.BlockSpec((1, 1), lambda i: (0, 0)),
        ],
        out_specs=pl.BlockSpec((BB, 1), lambda i: (i, 0)),
        out_shape=jax.ShapeDtypeStruct((B, 1), jnp.float32),
    )(idx, guess_table, constraint_table, W1, b1r, W2, b2r)


# speedup vs baseline: 73.0415x; 3.0201x over previous
import jax
import jax.numpy as jnp
from jax.experimental import pallas as pl

B = 16384
D = 8
H = 256
BB = 2048


def _body(idx_ref, gt_ref, ct_ref, w1_ref, b1_ref, w2_ref, b2_ref, out_ref):
    gpad = jnp.concatenate([gt_ref[:], jnp.zeros((6, D), jnp.float32)], axis=0)
    cpad = jnp.concatenate([ct_ref[:], jnp.zeros((5, D), jnp.float32)], axis=0)
    w1 = w1_ref[:]
    parts = []
    for p in range(5):
        parts.append(jax.lax.dot(gpad, w1[8 * p:8 * p + 8, :],
                                 preferred_element_type=jnp.float32))
    for p in range(5):
        parts.append(jax.lax.dot(cpad, w1[40 + 8 * p:48 + 8 * p, :],
                                 preferred_element_type=jnp.float32))
    parts.append(jax.lax.dot(gpad, w1[80:88, :], preferred_element_type=jnp.float32))
    parts.append(jax.lax.dot(gpad, w1[88:96, :], preferred_element_type=jnp.float32))
    a = jnp.concatenate(parts, axis=0)

    si = jax.lax.broadcasted_iota(jnp.int32, (32, BB), 0)
    tiles = []
    for s in range(10):
        row = idx_ref[s:s + 1, :]
        tiles.append(jnp.where(si == row, 1.0, 0.0))
    acc = jnp.zeros((32, BB), jnp.float32)
    for j in range(10):
        acc = acc + jnp.where(si == idx_ref[10 + j:11 + j, :], 0.1, 0.0)
    tiles.append(acc)
    acc = jnp.zeros((32, BB), jnp.float32)
    for j in range(10):
        acc = acc + jnp.where(si == idx_ref[20 + j:21 + j, :], 0.1, 0.0)
    tiles.append(acc)
    o_t = jnp.concatenate(tiles, axis=0)

    h = jax.lax.dot_general(o_t, a, (((0,), (0,)), ((), ())),
                            preferred_element_type=jnp.float32) + b1_ref[:]
    h = jnp.maximum(h, 0.0)
    out = jax.lax.dot(h, w2_ref[:], preferred_element_type=jnp.float32)
    out_ref[:] = out + b2_ref[:]


@jax.jit
def kernel(guess_indices, constraint_indices, presence_list, absent_list,
           guess_table, constraint_table, W1, b1, W2, b2):
    idx_t = jnp.concatenate([guess_indices, constraint_indices,
                             presence_list, absent_list], axis=1).astype(jnp.int32).T
    b1r = b1.reshape(1, H)
    b2r = b2.reshape(1, 1)
    grid = (B // BB,)
    return pl.pallas_call(
        _body,
        grid=grid,
        in_specs=[
            pl.BlockSpec((30, BB), lambda i: (0, i)),
            pl.BlockSpec((26, D), lambda i: (0, 0)),
            pl.BlockSpec((27, D), lambda i: (0, 0)),
            pl.BlockSpec((96, H), lambda i: (0, 0)),
            pl.BlockSpec((1, H), lambda i: (0, 0)),
            pl.BlockSpec((H, 1), lambda i: (0, 0)),
            pl.BlockSpec((1, 1), lambda i: (0, 0)),
        ],
        out_specs=pl.BlockSpec((BB, 1), lambda i: (i, 0)),
        out_shape=jax.ShapeDtypeStruct((B, 1), jnp.float32),
    )(idx_t, guess_table, constraint_table, W1, b1r, W2, b2r)
